# Initial kernel scaffold; baseline (speedup 1.0000x reference)
import jax
import jax.numpy as jnp
from jax.experimental import pallas as pl


def kernel(x_user, x_merchant, x_transaction, ei_user_tx, ei_tx_merchant, ei_tx_user, ei_merchant_tx, user_emb, merchant_emb, tx_lin_W, tx_lin_b, gat_W_src, gat_W_dst, gat_att_src, gat_att_dst, gat_bias, proj_W, proj_b):
    raise NotImplementedError("write your pallas kernel here")


# speedup vs baseline: 10.5306x; 10.5306x over previous
import jax
import jax.numpy as jnp
from jax import lax
from jax.experimental import pallas as pl
from jax.experimental.pallas import tpu as pltpu
from jax.experimental.pallas import tpu_sc as plsc

f32 = jnp.float32
i32 = jnp.int32

H = 4
C = 64
HID = 256
NOUT = 128
N = 20000
NP = 20224
BLK = 128
NBLK = NP // BLK
E = 50000
EP = 50176
BE = 112
NBE = EP // BE
NC = 2
NS = 16
NW = NC * NS
DBT = NBE // NS
EPT = DBT * BE
NP2 = NP // 2
ACCR = 10240
RPT = ACCR // NS
EMB_PT = NP // NW


def _mesh():
    return plsc.VectorSubcoreMesh(core_axis_name="c", subcore_axis_name="s",
                                  num_cores=NC, num_subcores=NS)


def _gather_lanes(vec, idx16):
    return lax.gather(
        vec, idx16.reshape(16, 1),
        lax.GatherDimensionNumbers(offset_dims=(),
                                   collapsed_slice_dims=(0,),
                                   start_index_map=(0,)),
        slice_sizes=(1,),
        mode=lax.GatherScatterMode.PROMISE_IN_BOUNDS)


def _bcast_lane(vec, lane):
    return _gather_lanes(vec, jnp.full((16,), lane, i32))


def _txlin(x, w, b):
    def body(x_ref, w_ref, b_ref, o_ref):
        o_ref[...] = jnp.maximum(
            jnp.dot(x_ref[...], w_ref[...], preferred_element_type=f32)
            + b_ref[...], 0.0)
    return pl.pallas_call(
        body,
        grid=(NBLK,),
        in_specs=[pl.BlockSpec((BLK, 64), lambda i: (i, 0)),
                  pl.BlockSpec((64, HID), lambda i: (0, 0)),
                  pl.BlockSpec((1, HID), lambda i: (0, 0))],
        out_specs=pl.BlockSpec((BLK, HID), lambda i: (i, 0)),
        out_shape=jax.ShapeDtypeStruct((NP, HID), f32),
    )(x, w, b)


def _conv_tc(xs, xd, w, a_src, a_dst):
    def body(xs_ref, xd_ref, w_ref, as_ref, ad_ref,
             h01, h23, oas, oad):
        x = xs_ref[...]
        hs = jnp.dot(x, w_ref[...], preferred_element_type=f32)
        h01[...] = hs[:, :128]
        h23[...] = hs[:, 128:]
        oas[...] = jnp.dot(x, as_ref[...], preferred_element_type=f32)
        oad[...] = jnp.dot(xd_ref[...], ad_ref[...],
                           preferred_element_type=f32)
    wide = jax.ShapeDtypeStruct((NP, 128), f32)
    return pl.pallas_call(
        body,
        grid=(NBLK,),
        in_specs=[pl.BlockSpec((BLK, HID), lambda i: (i, 0)),
                  pl.BlockSpec((BLK, HID), lambda i: (i, 0)),
                  pl.BlockSpec((HID, HID), lambda i: (0, 0)),
                  pl.BlockSpec((HID, 128), lambda i: (0, 0)),
                  pl.BlockSpec((HID, 128), lambda i: (0, 0))],
        out_specs=[pl.BlockSpec((BLK, 128), lambda i: (i, 0))] * 4,
        out_shape=[wide] * 4,
    )(xs, xd, w, a_src, a_dst)


def _combine(msgs, dens, bias, proj_w=None, proj_b=None):
    nconv = len(dens)
    final = proj_w is not None

    def body(*refs):
        m_refs = refs[:2 * nconv]
        den_refs = refs[2 * nconv:3 * nconv]
        b_ref = refs[3 * nconv]
        o_ref = refs[-1]
        cols = [None] * 4
        for v in range(nconv):
            den = den_refs[v][...]
            for p in range(2):
                m = m_refs[2 * v + p][...]
                for q in range(2):
                    h = 2 * p + q
                    t = m[:, q * C:(q + 1) * C] / (den[:, h:h + 1] + 1e-16)
                    cols[h] = t if cols[h] is None else cols[h] + t
        act = jnp.maximum(jnp.concatenate(cols, axis=1) + b_ref[...], 0.0)
        if final:
            pw_ref, pb_ref = refs[3 * nconv + 1:3 * nconv + 3]
            o_ref[...] = (jnp.dot(act, pw_ref[...],
                                  preferred_element_type=f32) + pb_ref[...])
        else:
            o_ref[...] = act

    in_specs = [pl.BlockSpec((BLK, 128), lambda i: (i, 0))] * (3 * nconv)
    in_specs.append(pl.BlockSpec((1, HID), lambda i: (0, 0)))
    flat_msgs = [m for pair in msgs for m in pair]
    args = flat_msgs + list(dens) + [bias]
    if final:
        in_specs += [pl.BlockSpec((HID, NOUT), lambda i: (0, 0)),
                     pl.BlockSpec((1, NOUT), lambda i: (0, 0))]
        args += [proj_w, proj_b]
        out_spec = pl.BlockSpec((BLK, NOUT), lambda i: (i, 0))
        out_shape = jax.ShapeDtypeStruct((NP, NOUT), f32)
    else:
        out_spec = pl.BlockSpec((BLK, HID), lambda i: (i, 0))
        out_shape = jax.ShapeDtypeStruct((NP, HID), f32)
    return pl.pallas_call(
        body, grid=(NBLK,), in_specs=in_specs, out_specs=out_spec,
        out_shape=out_shape)(*args)


def _emb_gather(emb, idx2d):
    def body(emb_hbm, idx_hbm, out_hbm, idxv, rows, sem):
        cid = lax.axis_index("c")
        sid = lax.axis_index("s")
        wid = sid * NC + cid
        base = wid * EMB_PT
        for k in range(5):
            sz = 128 if k < 4 else EMB_PT - 512
            pltpu.sync_copy(idx_hbm.at[wid * 5 + k], idxv)
            pltpu.async_copy(emb_hbm.at[idxv], rows, sem).wait()
            pltpu.sync_copy(rows.at[pl.ds(0, sz)],
                            out_hbm.at[pl.ds(base + k * 128, sz)])
    return pl.kernel(
        body,
        out_type=jax.ShapeDtypeStruct((NP, HID), f32),
        mesh=_mesh(),
        scratch_types=[pltpu.VMEM((128,), i32),
                       pltpu.VMEM((128, HID), f32),
                       pltpu.SemaphoreType.DMA],
    )(emb, idx2d)


def _gat_edge_sc(a_s, a_d, src1d, dst1d, hs01, hs23):
    def body(as_hbm, ad_hbm, src_hbm, dst_hbm, h01_hbm, h23_hbm,
             den_hbm, m01_hbm, m23_hbm,
             sb, db, dlb, exv, ga, gb, zb, acc_sh, sem, sem2):
        cid = lax.axis_index("c")
        sid = lax.axis_index("s")
        io16 = lax.iota(i32, 16)

        def zrow(i, carry):
            for q in range(8):
                zb[i, q * 16:(q + 1) * 16] = jnp.zeros((16,), f32)
            return carry
        lax.fori_loop(0, 32, zrow, 0)

        def remap(k, carry2):
            v = db[pl.ds(k * 16, 16)] - cid * NP2
            ok = (v >= 0) & (v < NP2)
            dlb[pl.ds(k * 16, 16)] = jnp.where(ok, v, NP2)
            return carry2

        for t in range(20):
            pltpu.sync_copy(zb, acc_sh.at[pl.ds(sid * RPT + t * 32, 32)])
        plsc.subcore_barrier()

        def den_blk(j, carry):
            boff = (DBT * sid + j) * BE
            pltpu.sync_copy(src_hbm.at[pl.ds(boff, BE)], sb)
            pltpu.sync_copy(dst_hbm.at[pl.ds(boff, BE)], db)
            cp1 = pltpu.async_copy(as_hbm.at[sb], ga, sem)
            cp2 = pltpu.async_copy(ad_hbm.at[db], gb, sem2)
            cp1.wait()
            cp2.wait()
            lax.fori_loop(0, BE // 16, remap, 0)

            def e4(i4, carry2):
                exs = []
                for r in range(4):
                    i = i4 * 4 + r
                    v = ga[i, 0:16] + gb[i, 0:16]
                    v = jnp.maximum(v, v * 0.2)
                    ex = jnp.exp(v)
                    gb[i, 0:16] = ex
                    exs.append(ex)
                packed = _gather_lanes(exs[0], jnp.clip(io16, 0, 3))
                for r in range(1, 4):
                    g = _gather_lanes(exs[r], jnp.clip(io16 - 4 * r, 0, 3))
                    packed = jnp.where((io16 >> 2) == r, g, packed)
                exv[pl.ds((j * (BE // 4) + i4) * 16, 16)] = packed
                return carry2
            lax.fori_loop(0, BE // 4, e4, 0)
            pltpu.sync_copy(gb, acc_sh.at[dlb], add=True)
            return carry
        lax.fori_loop(0, DBT, den_blk, 0)
        plsc.subcore_barrier()

        @pl.when(sid < NS - 1)
        def _():
            pltpu.sync_copy(acc_sh.at[pl.ds(sid * RPT, RPT)],
                            den_hbm.at[pl.ds(cid * NP2 + sid * RPT, RPT)])

        @pl.when(sid == NS - 1)
        def _():
            pltpu.sync_copy(acc_sh.at[pl.ds((NS - 1) * RPT, NP2 - (NS - 1) * RPT)],
                            den_hbm.at[pl.ds(cid * NP2 + (NS - 1) * RPT,
                                             NP2 - (NS - 1) * RPT)])
        plsc.subcore_barrier()

        for p, (hs_hbm, out_hbm) in enumerate(((h01_hbm, m01_hbm),
                                               (h23_hbm, m23_hbm))):
            for t in range(20):
                pltpu.sync_copy(zb, acc_sh.at[pl.ds(sid * RPT + t * 32, 32)])
            plsc.subcore_barrier()

            def msg_blk(j, carry):
                boff = (DBT * sid + j) * BE
                pltpu.sync_copy(src_hbm.at[pl.ds(boff, BE)], sb)
                pltpu.sync_copy(dst_hbm.at[pl.ds(boff, BE)], db)
                pltpu.async_copy(hs_hbm.at[sb], ga, sem).wait()
                lax.fori_loop(0, BE // 16, remap, 0)

                def mrow(i, carry2):
                    g = j * BE + i
                    e16 = exv[pl.ds((g >> 2) * 16, 16)]
                    lane = (g & 3) * 4 + 2 * p
                    s0 = _bcast_lane(e16, lane)
                    s1 = _bcast_lane(e16, lane + 1)
                    for q in range(8):
                        s = s0 if q < 4 else s1
                        gb[i, q * 16:(q + 1) * 16] = (
                            ga[i, q * 16:(q + 1) * 16] * s)
                    return carry2
                lax.fori_loop(0, BE, mrow, 0)
                pltpu.sync_copy(gb, acc_sh.at[dlb], add=True)
                return carry
            lax.fori_loop(0, DBT, msg_blk, 0)
            plsc.subcore_barrier()

            @pl.when(sid < NS - 1)
            def _():
                pltpu.sync_copy(acc_sh.at[pl.ds(sid * RPT, RPT)],
                                out_hbm.at[pl.ds(cid * NP2 + sid * RPT, RPT)])

            @pl.when(sid == NS - 1)
            def _():
                pltpu.sync_copy(
                    acc_sh.at[pl.ds((NS - 1) * RPT, NP2 - (NS - 1) * RPT)],
                    out_hbm.at[pl.ds(cid * NP2 + (NS - 1) * RPT,
                                     NP2 - (NS - 1) * RPT)])
            plsc.subcore_barrier()

    wide = jax.ShapeDtypeStruct((NP, 128), f32)
    return pl.kernel(
        body,
        out_type=(wide, wide, wide),
        mesh=_mesh(),
        scratch_types=[pltpu.VMEM((BE,), i32),
                       pltpu.VMEM((BE,), i32),
                       pltpu.VMEM((BE,), i32),
                       pltpu.VMEM((EPT * 4,), f32),
                       pltpu.VMEM((BE, 128), f32),
                       pltpu.VMEM((BE, 128), f32),
                       pltpu.VMEM((32, 128), f32),
                       pltpu.VMEM_SHARED((ACCR, 128), f32),
                       pltpu.SemaphoreType.DMA,
                       pltpu.SemaphoreType.DMA],
    )(a_s, a_d, src1d, dst1d, hs01, hs23)


def kernel(x_user, x_merchant, x_transaction, ei_user_tx, ei_tx_merchant,
           ei_tx_user, ei_merchant_tx, user_emb, merchant_emb, tx_lin_W,
           tx_lin_b, gat_W_src, gat_W_dst, gat_att_src, gat_att_dst,
           gat_bias, proj_W, proj_b):
    def prep_idx(x):
        xp = jnp.pad(x.astype(i32), (0, NP - N))
        xp = jnp.pad(xp.reshape(NW, EMB_PT), ((0, 0), (0, 8)))
        return xp.reshape(NW * 5, 128)

    def prep_edges(ei):
        s = jnp.pad(ei[0].astype(i32), (0, EP - E))
        d = jnp.pad(ei[1].astype(i32), (0, EP - E), constant_values=N)
        return s, d

    h_u = _emb_gather(user_emb, prep_idx(x_user))
    h_m = _emb_gather(merchant_emb, prep_idx(x_merchant))
    h_t = _txlin(jnp.pad(x_transaction, ((0, NP - N), (0, 0))),
                 tx_lin_W, tx_lin_b.reshape(1, HID))

    e_ut = prep_edges(ei_user_tx)
    e_tm = prep_edges(ei_tx_merchant)
    e_tu = prep_edges(ei_tx_user)
    e_mt = prep_edges(ei_merchant_tx)

    a_src_all = jnp.einsum('lekhc,lehc->lekh',
                           gat_W_src.reshape(2, 4, HID, H, C), gat_att_src)
    a_dst_all = jnp.einsum('lekhc,lehc->lekh',
                           gat_W_dst.reshape(2, 4, HID, H, C), gat_att_dst)
    a_src_all = jnp.pad(a_src_all, ((0, 0), (0, 0), (0, 0), (0, 124)))
    a_dst_all = jnp.pad(a_dst_all, ((0, 0), (0, 0), (0, 0), (0, 124)))

    def conv(xs, xd, l, e, edges):
        hs01, hs23, a_s, a_d = _conv_tc(
            xs, xd, gat_W_src[l, e], a_src_all[l, e], a_dst_all[l, e])
        s1d, d1d = edges
        den, m01, m23 = _gat_edge_sc(a_s, a_d, s1d, d1d, hs01, hs23)
        return den, (m01, m23)

    d_ut, m_ut = conv(h_u, h_t, 0, 0, e_ut)
    d_tm, m_tm = conv(h_t, h_m, 0, 1, e_tm)
    d_tu, m_tu = conv(h_t, h_u, 0, 2, e_tu)
    d_mt, m_mt = conv(h_m, h_t, 0, 3, e_mt)
    h_t1 = _combine([m_ut, m_mt], [d_ut, d_mt],
                    (gat_bias[0, 0] + gat_bias[0, 3]).reshape(1, HID))
    h_m1 = _combine([m_tm], [d_tm], gat_bias[0, 1].reshape(1, HID))
    h_u1 = _combine([m_tu], [d_tu], gat_bias[0, 2].reshape(1, HID))

    d_ut2, m_ut2 = conv(h_u1, h_t1, 1, 0, e_ut)
    d_mt2, m_mt2 = conv(h_m1, h_t1, 1, 3, e_mt)
    out = _combine([m_ut2, m_mt2], [d_ut2, d_mt2],
                   (gat_bias[1, 0] + gat_bias[1, 3]).reshape(1, HID),
                   proj_W, proj_b.reshape(1, NOUT))
    return out[:N]
